# baseline (speedup 1.0000x reference)
import functools

import jax
import jax.numpy as jnp
from jax import lax
from jax.experimental import pallas as pl
from jax.experimental.pallas import tpu as pltpu
from jax.experimental.pallas import tpu_sc as plsc

_B = 16384
_NF = 26
_FV = 100000
_E = 16
_IN = (_NF + 2) * _E
_NC = 2
_NS = 16
_NW = _NC * _NS
_BPT = _B // _NW
_CH = 128
_NCH = _BPT // _CH
_BLK = 2048


def _sc_gather(user_ids, item_ids, sfT, user_table, item_table, sparse_tables):
    mesh = plsc.VectorSubcoreMesh(core_axis_name="c", subcore_axis_name="s")

    @functools.partial(
        pl.kernel,
        out_type=jax.ShapeDtypeStruct((_B, _IN), jnp.float32),
        mesh=mesh,
        scratch_types=[
            pltpu.VMEM((_CH,), jnp.int32),
            pltpu.VMEM((_CH,), jnp.int32),
            pltpu.VMEM((_NF, _CH), jnp.int32),
            pltpu.VMEM((_CH, _E), jnp.float32),
            pltpu.VMEM((_CH, _E), jnp.float32),
            pltpu.VMEM((_NF, _CH, _E), jnp.float32),
            pltpu.SemaphoreType.DMA,
        ],
        compiler_params=pltpu.CompilerParams(use_tc_tiling_on_sc=False),
    )
    def k(uids, iids, sft, ut, itb, st, out,
          uidx, iidx, sidx, urows, irows, srows, sem):
        wid = lax.axis_index("s") * _NC + lax.axis_index("c")
        base = wid * _BPT
        for c in range(_NCH):
            gb = base + c * _CH
            pltpu.sync_copy(uids.at[pl.ds(gb, _CH)], uidx)
            pltpu.sync_copy(iids.at[pl.ds(gb, _CH)], iidx)
            pltpu.sync_copy(sft.at[:, pl.ds(gb, _CH)], sidx)
            cps = [
                pltpu.async_copy(ut.at[uidx], urows, sem),
                pltpu.async_copy(itb.at[iidx], irows, sem),
            ]
            for f in range(_NF):
                cps.append(pltpu.async_copy(
                    st.at[pl.ds(f * _FV, _FV)].at[sidx.at[f]], srows.at[f], sem))
            for cp in cps:
                cp.wait()
            pltpu.sync_copy(urows, out.at[pl.ds(gb, _CH), pl.ds(0, _E)])
            pltpu.sync_copy(irows, out.at[pl.ds(gb, _CH), pl.ds(_E, _E)])
            for f in range(_NF):
                pltpu.sync_copy(srows.at[f],
                                out.at[pl.ds(gb, _CH), pl.ds((2 + f) * _E, _E)])

    return k(user_ids, item_ids, sfT, user_table, item_table, sparse_tables)


def _tc_body(x_ref, w1_ref, b1_ref, w2_ref, b2_ref, w3_ref, b3_ref,
             w4_ref, b4_ref, o_ref):
    x = x_ref[...]
    ssum = jnp.sum(x, axis=1, keepdims=True)
    ssq = jnp.sum(x * x, axis=1, keepdims=True)
    h = jnp.maximum(jnp.dot(x, w1_ref[...], preferred_element_type=jnp.float32)
                    + b1_ref[...], 0.0)
    h = jnp.maximum(jnp.dot(h, w2_ref[...], preferred_element_type=jnp.float32)
                    + b2_ref[...], 0.0)
    h = jnp.maximum(jnp.dot(h, w3_ref[...], preferred_element_type=jnp.float32)
                    + b3_ref[...], 0.0)
    deep = jnp.sum(h * w4_ref[...], axis=1, keepdims=True) + b4_ref[...]
    fm = 0.5 * (ssum * ssum - ssq)
    o_ref[...] = jax.nn.sigmoid(deep + fm)


def _tc_mlp(x, w1, b1, w2, b2, w3, b3, w4r, b4):
    full = lambda i: (0, 0)
    return pl.pallas_call(
        _tc_body,
        grid=(_B // _BLK,),
        in_specs=[
            pl.BlockSpec((_BLK, _IN), lambda i: (i, 0)),
            pl.BlockSpec((_IN, _E), full),
            pl.BlockSpec((1, _E), full),
            pl.BlockSpec((_E, _E), full),
            pl.BlockSpec((1, _E), full),
            pl.BlockSpec((_E, _E), full),
            pl.BlockSpec((1, _E), full),
            pl.BlockSpec((1, _E), full),
            pl.BlockSpec((1, 1), full),
        ],
        out_specs=pl.BlockSpec((_BLK, 1), lambda i: (i, 0)),
        out_shape=jax.ShapeDtypeStruct((_B, 1), jnp.float32),
    )(x, w1, b1, w2, b2, w3, b3, w4r, b4)


def _sc_depad(t, rows):
    nch = rows // 320
    mesh = plsc.VectorSubcoreMesh(core_axis_name="c", subcore_axis_name="s")

    @functools.partial(
        pl.kernel,
        out_type=jax.ShapeDtypeStruct((rows // 8, 128), jnp.float32),
        mesh=mesh,
        scratch_types=[
            pltpu.VMEM((320, _E), jnp.float32),
            pltpu.VMEM((40, 128), jnp.float32),
        ],
        compiler_params=pltpu.CompilerParams(use_tc_tiling_on_sc=True),
    )
    def k(th, out, src, dst):
        wid = lax.axis_index("s") * _NC + lax.axis_index("c")

        def chunk_body(i, _):
            c = i * _NW + wid

            @pl.when(c < nch)
            def _():
                pltpu.sync_copy(
                    th.at[pl.ds(pl.multiple_of(c * 320, 8), 320)], src)

                def move(j, _):
                    for s in range(8):
                        dst[j, pl.ds(s * _E, _E)] = src[8 * j + s, :]
                    return ()

                lax.fori_loop(0, 40, move, ())
                pltpu.sync_copy(
                    dst, out.at[pl.ds(pl.multiple_of(c * 40, 8), 40), :])
            return ()

        lax.fori_loop(0, (nch + _NW - 1) // _NW, chunk_body, ())

    return k(t).reshape(rows, _E)


def _row_major(t, rows):
    return _sc_depad(jnp.reshape(t, (rows, _E)), rows)


def kernel(user_ids, item_ids, sparse_features, user_table, item_table,
           sparse_tables, W1, b1, W2, b2, W3, b3, W4, b4):
    sfT = sparse_features.T
    comb = _sc_gather(user_ids, item_ids, sfT,
                      _row_major(user_table, 1000000),
                      _row_major(item_table, 1000000),
                      _row_major(sparse_tables, _NF * _FV))
    out = _tc_mlp(comb, W1, b1.reshape(1, _E), W2, b2.reshape(1, _E),
                  W3, b3.reshape(1, _E), W4.reshape(1, _E), b4.reshape(1, 1))
    return out.reshape(_B)


# speedup vs baseline: 1.2354x; 1.2354x over previous
import functools

import jax
import jax.numpy as jnp
from jax import lax
from jax.experimental import pallas as pl
from jax.experimental.pallas import tpu as pltpu
from jax.experimental.pallas import tpu_sc as plsc

_B = 16384
_NF = 26
_FV = 100000
_E = 16
_IN = (_NF + 2) * _E
_NC = 2
_NS = 16
_NW = _NC * _NS
_BPT = _B // _NW
_CH = 128
_NCH = _BPT // _CH
_BLK = 2048


def _sc_gather(user_ids, item_ids, sfT, user_table, item_table, sparse_tables):
    mesh = plsc.VectorSubcoreMesh(core_axis_name="c", subcore_axis_name="s")

    @functools.partial(
        pl.kernel,
        out_type=jax.ShapeDtypeStruct((_B, _IN), jnp.float32),
        mesh=mesh,
        scratch_types=[
            pltpu.VMEM((_CH,), jnp.int32),
            pltpu.VMEM((_CH,), jnp.int32),
            pltpu.VMEM((_NF, _CH), jnp.int32),
            pltpu.VMEM((_CH, _E), jnp.float32),
            pltpu.VMEM((_CH, _E), jnp.float32),
            pltpu.VMEM((_NF, _CH, _E), jnp.float32),
            pltpu.SemaphoreType.DMA,
        ],
        compiler_params=pltpu.CompilerParams(use_tc_tiling_on_sc=False),
    )
    def k(uids, iids, sft, ut, itb, st, out,
          uidx, iidx, sidx, urows, irows, srows, sem):
        wid = lax.axis_index("s") * _NC + lax.axis_index("c")
        base = wid * _BPT
        for c in range(_NCH):
            gb = base + c * _CH
            pltpu.sync_copy(uids.at[pl.ds(gb, _CH)], uidx)
            pltpu.sync_copy(iids.at[pl.ds(gb, _CH)], iidx)
            pltpu.sync_copy(sft.at[:, pl.ds(gb, _CH)], sidx)
            cps = [
                pltpu.async_copy(ut.at[uidx], urows, sem),
                pltpu.async_copy(itb.at[iidx], irows, sem),
            ]
            for f in range(_NF):
                cps.append(pltpu.async_copy(
                    st.at[pl.ds(f * _FV, _FV)].at[sidx.at[f]], srows.at[f], sem))
            for cp in cps:
                cp.wait()
            pltpu.sync_copy(urows, out.at[pl.ds(gb, _CH), pl.ds(0, _E)])
            pltpu.sync_copy(irows, out.at[pl.ds(gb, _CH), pl.ds(_E, _E)])
            for f in range(_NF):
                pltpu.sync_copy(srows.at[f],
                                out.at[pl.ds(gb, _CH), pl.ds((2 + f) * _E, _E)])

    return k(user_ids, item_ids, sfT, user_table, item_table, sparse_tables)


def _tc_body(x_ref, w1_ref, b1_ref, w2_ref, b2_ref, w3_ref, b3_ref,
             w4_ref, b4_ref, o_ref):
    x = x_ref[...]
    ssum = jnp.sum(x, axis=1, keepdims=True)
    ssq = jnp.sum(x * x, axis=1, keepdims=True)
    h = jnp.maximum(jnp.dot(x, w1_ref[...], preferred_element_type=jnp.float32)
                    + b1_ref[...], 0.0)
    h = jnp.maximum(jnp.dot(h, w2_ref[...], preferred_element_type=jnp.float32)
                    + b2_ref[...], 0.0)
    h = jnp.maximum(jnp.dot(h, w3_ref[...], preferred_element_type=jnp.float32)
                    + b3_ref[...], 0.0)
    deep = jnp.sum(h * w4_ref[...], axis=1, keepdims=True) + b4_ref[...]
    fm = 0.5 * (ssum * ssum - ssq)
    o_ref[...] = jax.nn.sigmoid(deep + fm)


def _tc_mlp(x, w1, b1, w2, b2, w3, b3, w4r, b4):
    full = lambda i: (0, 0)
    return pl.pallas_call(
        _tc_body,
        grid=(_B // _BLK,),
        in_specs=[
            pl.BlockSpec((_BLK, _IN), lambda i: (i, 0)),
            pl.BlockSpec((_IN, _E), full),
            pl.BlockSpec((1, _E), full),
            pl.BlockSpec((_E, _E), full),
            pl.BlockSpec((1, _E), full),
            pl.BlockSpec((_E, _E), full),
            pl.BlockSpec((1, _E), full),
            pl.BlockSpec((1, _E), full),
            pl.BlockSpec((1, 1), full),
        ],
        out_specs=pl.BlockSpec((_BLK, 1), lambda i: (i, 0)),
        out_shape=jax.ShapeDtypeStruct((_B, 1), jnp.float32),
    )(x, w1, b1, w2, b2, w3, b3, w4r, b4)


def _row_major(t, rows):
    t128 = jax.lax.optimization_barrier(jnp.reshape(t, (rows // 8, 128)))
    return t128.reshape(rows, _E)


def kernel(user_ids, item_ids, sparse_features, user_table, item_table,
           sparse_tables, W1, b1, W2, b2, W3, b3, W4, b4):
    sfT = sparse_features.T
    comb = _sc_gather(user_ids, item_ids, sfT,
                      _row_major(user_table, 1000000),
                      _row_major(item_table, 1000000),
                      _row_major(sparse_tables, _NF * _FV))
    out = _tc_mlp(comb, W1, b1.reshape(1, _E), W2, b2.reshape(1, _E),
                  W3, b3.reshape(1, _E), W4.reshape(1, _E), b4.reshape(1, 1))
    return out.reshape(_B)
